# baseline (speedup 1.0000x reference)
import functools

import jax
import jax.numpy as jnp
from jax import lax
from jax.experimental import pallas as pl
from jax.experimental.pallas import tpu as pltpu
from jax.experimental.pallas import tpu_sc as plsc

_B, _H, _W, _F = 16, 32, 32, 256
_K = 2


def _sc_body(row_hbm, col_hbm, out_hbm, tile_v, sem_in, sem_out):
    wid = lax.axis_index("s") * 2 + lax.axis_index("c")
    r = wid
    copies = [
        pltpu.make_async_copy(col_hbm.at[pl.ds(0, _W)], tile_v.at[:, pl.ds(0, _F)], sem_in)
    ]
    for i in range(_W):
        copies.append(
            pltpu.make_async_copy(
                row_hbm.at[pl.ds(r, 1)], tile_v.at[pl.ds(i, 1), pl.ds(_F, _F)], sem_in
            )
        )
    for c in copies:
        c.start()
    for c in copies:
        c.wait()
    outs = [
        pltpu.make_async_copy(
            tile_v, out_hbm.at[pl.ds((b * _H + r) * _W, _W)], sem_out
        )
        for b in range(_K)
    ]
    for c in outs:
        c.start()
    for c in outs:
        c.wait()


def _tc_body(donated_ref, row_ref, col_ref, out_ref, slab, sem):
    del donated_ref
    col = col_ref[0:_W, :]
    row = row_ref[0:_H, :]
    x_part = jnp.broadcast_to(col[None, None, :, :], (1, _H, _W, _F))
    y_part = jnp.broadcast_to(row[None, :, None, :], (1, _H, _W, _F))
    slab[...] = jnp.concatenate([x_part, y_part], axis=-1)
    copies = [
        pltpu.make_async_copy(slab, out_ref.at[pl.ds(b, 1)], sem)
        for b in range(_K, _B)
    ]
    for c in copies:
        c.start()
    for c in copies:
        c.wait()


def kernel(img, row_embed, col_embed):
    del img
    mesh = plsc.VectorSubcoreMesh(core_axis_name="c", subcore_axis_name="s")
    sc = functools.partial(
        pl.kernel,
        mesh=mesh,
        out_type=jax.ShapeDtypeStruct((_B * _H * _W, 2 * _F), jnp.float32),
        scratch_types=[
            pltpu.VMEM((_W, 2 * _F), jnp.float32),
            pltpu.SemaphoreType.DMA,
            pltpu.SemaphoreType.DMA,
        ],
    )(_sc_body)
    sc_out = sc(row_embed, col_embed).reshape(_B, _H, _W, 2 * _F)
    return pl.pallas_call(
        _tc_body,
        in_specs=[
            pl.BlockSpec(memory_space=pl.ANY),
            pl.BlockSpec(memory_space=pltpu.VMEM),
            pl.BlockSpec(memory_space=pltpu.VMEM),
        ],
        out_specs=pl.BlockSpec(memory_space=pl.ANY),
        out_shape=jax.ShapeDtypeStruct((_B, _H, _W, 2 * _F), jnp.float32),
        scratch_shapes=[
            pltpu.VMEM((1, _H, _W, 2 * _F), jnp.float32),
            pltpu.SemaphoreType.DMA,
        ],
        input_output_aliases={0: 0},
    )(sc_out, row_embed, col_embed)


# speedup vs baseline: 3.1510x; 3.1510x over previous
import jax
import jax.numpy as jnp
from jax.experimental import pallas as pl
from jax.experimental.pallas import tpu as pltpu

_B, _H, _W, _F = 16, 32, 32, 256


def _pos_body(row_ref, col_ref, out_ref, slab, sem):
    col = col_ref[0:_W, :]
    row = row_ref[0:_H, :]
    x_part = jnp.broadcast_to(col[None, None, :, :], (1, _H, _W, _F))
    y_part = jnp.broadcast_to(row[None, :, None, :], (1, _H, _W, _F))
    slab[...] = jnp.concatenate([x_part, y_part], axis=-1)
    copies = [
        pltpu.make_async_copy(
            slab.at[:, pl.ds(h, _H // 2)],
            out_ref.at[pl.ds(b, 1), pl.ds(h, _H // 2)],
            sem,
        )
        for b in range(_B)
        for h in (0, _H // 2)
    ]
    for c in copies:
        c.start()
    for c in copies:
        c.wait()


def kernel(img, row_embed, col_embed):
    del img
    out_shape = jax.ShapeDtypeStruct((_B, _H, _W, 2 * _F), jnp.float32)
    return pl.pallas_call(
        _pos_body,
        in_specs=[
            pl.BlockSpec(memory_space=pltpu.VMEM),
            pl.BlockSpec(memory_space=pltpu.VMEM),
        ],
        out_specs=pl.BlockSpec(memory_space=pl.ANY),
        out_shape=out_shape,
        scratch_shapes=[
            pltpu.VMEM((1, _H, _W, 2 * _F), jnp.float32),
            pltpu.SemaphoreType.DMA,
        ],
    )(row_embed, col_embed)
